# baseline (speedup 1.0000x reference)
import jax
import jax.numpy as jnp
from jax.experimental import pallas as pl
from jax.experimental.pallas import tpu as pltpu

FEATURE_DIM = 1024
HIDDEN_DIM = 256
NUM_EXPERTS = 64
TOP_K = 8
STALENESS_LAMBDA = 0.005
STALENESS_FLOOR = 0.1
N_TOKENS = 16384

BLOCK_B = 1024


def _router_kernel(f_ref, w_ref, b_ref, emb_ref, ef_ref, tr_ref, dt_ref,
                   out_ref):
    ts = tr_ref[:] * jnp.maximum(
        jnp.exp(-STALENESS_LAMBDA * dt_ref[:]), STALENESS_FLOOR)
    f = f_ref[:]
    fb = f.astype(jnp.bfloat16)
    h = jnp.dot(fb, w_ref[:].astype(jnp.bfloat16),
                preferred_element_type=jnp.float32) + b_ref[:]
    logits_t = jax.lax.dot_general(
        emb_ref[:].astype(jnp.bfloat16), h.astype(jnp.bfloat16),
        (((1,), (1,)), ((), ())),
        preferred_element_type=jnp.float32)
    p = jnp.exp(logits_t)

    ef = ef_ref[:]
    en = ef / (jnp.sqrt(jnp.sum(ef * ef, axis=1, keepdims=True)) + 1e-8)
    rnorm = jnp.sqrt(jnp.sum(f * f, axis=1, keepdims=True)) + 1e-8
    fn = f / rnorm
    sim = jnp.maximum(jax.lax.dot_general(
        en.astype(jnp.bfloat16), fn.astype(jnp.bfloat16),
        (((1,), (1,)), ((), ())),
        preferred_element_type=jnp.float32), 0.0)

    scores = (p * sim) * ts

    work = scores
    for _ in range(TOP_K - 1):
        mx = jnp.max(work, axis=0, keepdims=True)
        work = jnp.where(work >= mx, -1.0, work)
    kth = jnp.max(work, axis=0, keepdims=True)

    masked = jnp.where(scores >= kth, scores, 0.0)
    w = masked / (jnp.sum(masked, axis=0, keepdims=True) + 1e-9)
    out_ref[:] = w.T


@jax.jit
def kernel(features, W_proj, b_proj, expert_emb, expert_features, trust,
           staleness_dt):
    tr2 = trust.reshape(NUM_EXPERTS, 1)
    dt2 = staleness_dt.reshape(NUM_EXPERTS, 1)
    b2 = b_proj.reshape(1, HIDDEN_DIM)
    n_blocks = N_TOKENS // BLOCK_B
    return pl.pallas_call(
        _router_kernel,
        grid=(n_blocks,),
        in_specs=[
            pl.BlockSpec((BLOCK_B, FEATURE_DIM), lambda i: (i, 0)),
            pl.BlockSpec((FEATURE_DIM, HIDDEN_DIM), lambda i: (0, 0)),
            pl.BlockSpec((1, HIDDEN_DIM), lambda i: (0, 0)),
            pl.BlockSpec((NUM_EXPERTS, HIDDEN_DIM), lambda i: (0, 0)),
            pl.BlockSpec((NUM_EXPERTS, FEATURE_DIM), lambda i: (0, 0)),
            pl.BlockSpec((NUM_EXPERTS, 1), lambda i: (0, 0)),
            pl.BlockSpec((NUM_EXPERTS, 1), lambda i: (0, 0)),
        ],
        out_specs=pl.BlockSpec((BLOCK_B, NUM_EXPERTS), lambda i: (i, 0)),
        out_shape=jax.ShapeDtypeStruct((N_TOKENS, NUM_EXPERTS), jnp.float32),
        compiler_params=pltpu.CompilerParams(
            dimension_semantics=("parallel",)),
    )(features, W_proj, b2, expert_emb, expert_features, tr2, dt2)


# speedup vs baseline: 1.0289x; 1.0289x over previous
import jax
import jax.numpy as jnp
from jax.experimental import pallas as pl
from jax.experimental.pallas import tpu as pltpu

FEATURE_DIM = 1024
HIDDEN_DIM = 256
NUM_EXPERTS = 64
TOP_K = 8
STALENESS_LAMBDA = 0.005
STALENESS_FLOOR = 0.1
N_TOKENS = 16384

BLOCK_B = 1024


def _router_kernel(f_ref, w_ref, b_ref, emb_ref, ef_ref, ts_ref, out_ref):
    ts = ts_ref[:]
    f = f_ref[:]
    fb = f.astype(jnp.bfloat16)
    h = jnp.dot(fb, w_ref[:].astype(jnp.bfloat16),
                preferred_element_type=jnp.float32) + b_ref[:]
    logits_t = jax.lax.dot_general(
        emb_ref[:].astype(jnp.bfloat16), h.astype(jnp.bfloat16),
        (((1,), (1,)), ((), ())),
        preferred_element_type=jnp.float32)
    p = jnp.exp(logits_t)

    ef = ef_ref[:]
    en = ef / (jnp.sqrt(jnp.sum(ef * ef, axis=1, keepdims=True)) + 1e-8)
    rnorm = jnp.sqrt(jnp.sum(f * f, axis=1, keepdims=True)) + 1e-8
    fn = f / rnorm
    sim = jnp.maximum(jax.lax.dot_general(
        en.astype(jnp.bfloat16), fn.astype(jnp.bfloat16),
        (((1,), (1,)), ((), ())),
        preferred_element_type=jnp.float32), 0.0)

    scores = (p * sim) * ts

    work = scores
    for _ in range(TOP_K - 1):
        mx = jnp.max(work, axis=0, keepdims=True)
        work = jnp.where(work >= mx, -1.0, work)
    kth = jnp.max(work, axis=0, keepdims=True)

    masked = jnp.where(scores >= kth, scores, 0.0)
    w = masked / (jnp.sum(masked, axis=0, keepdims=True) + 1e-9)
    out_ref[:] = w.T


@jax.jit
def kernel(features, W_proj, b_proj, expert_emb, expert_features, trust,
           staleness_dt):
    stale = jnp.maximum(jnp.exp(-STALENESS_LAMBDA * staleness_dt),
                        STALENESS_FLOOR)
    ts2 = (trust * stale).reshape(NUM_EXPERTS, 1)
    b2 = b_proj.reshape(1, HIDDEN_DIM)
    n_blocks = N_TOKENS // BLOCK_B
    return pl.pallas_call(
        _router_kernel,
        grid=(n_blocks,),
        in_specs=[
            pl.BlockSpec((BLOCK_B, FEATURE_DIM), lambda i: (i, 0)),
            pl.BlockSpec((FEATURE_DIM, HIDDEN_DIM), lambda i: (0, 0)),
            pl.BlockSpec((1, HIDDEN_DIM), lambda i: (0, 0)),
            pl.BlockSpec((NUM_EXPERTS, HIDDEN_DIM), lambda i: (0, 0)),
            pl.BlockSpec((NUM_EXPERTS, FEATURE_DIM), lambda i: (0, 0)),
            pl.BlockSpec((NUM_EXPERTS, 1), lambda i: (0, 0)),
        ],
        out_specs=pl.BlockSpec((BLOCK_B, NUM_EXPERTS), lambda i: (i, 0)),
        out_shape=jax.ShapeDtypeStruct((N_TOKENS, NUM_EXPERTS), jnp.float32),
        compiler_params=pltpu.CompilerParams(
            dimension_semantics=("parallel",)),
    )(features, W_proj, b2, expert_emb, expert_features, ts2)


# speedup vs baseline: 1.0771x; 1.0469x over previous
import jax
import jax.numpy as jnp
from jax.experimental import pallas as pl
from jax.experimental.pallas import tpu as pltpu

FEATURE_DIM = 1024
HIDDEN_DIM = 256
NUM_EXPERTS = 64
TOP_K = 8
STALENESS_LAMBDA = 0.005
STALENESS_FLOOR = 0.1
N_TOKENS = 16384

BLOCK_B = 1024


def _router_kernel(f_ref, w_ref, b_ref, emb_ref, ef_ref, ts_ref, out_ref):
    ts = ts_ref[:]
    half = BLOCK_B // 2
    for hi in range(2):
        _router_half(f_ref[pl.ds(hi * half, half), :], w_ref, b_ref,
                     emb_ref, ef_ref, ts, out_ref, hi * half)


def _router_half(f, w_ref, b_ref, emb_ref, ef_ref, ts, out_ref, row0):
    fb = f.astype(jnp.bfloat16)
    h = jnp.dot(fb, w_ref[:].astype(jnp.bfloat16),
                preferred_element_type=jnp.float32) + b_ref[:]
    logits_t = jax.lax.dot_general(
        emb_ref[:].astype(jnp.bfloat16), h.astype(jnp.bfloat16),
        (((1,), (1,)), ((), ())),
        preferred_element_type=jnp.float32)
    p = jnp.exp(logits_t)

    ef = ef_ref[:]
    en = ef / (jnp.sqrt(jnp.sum(ef * ef, axis=1, keepdims=True)) + 1e-8)
    rnorm = jnp.sqrt(jnp.sum(f * f, axis=1, keepdims=True)) + 1e-8
    fn = f / rnorm
    sim = jnp.maximum(jax.lax.dot_general(
        en.astype(jnp.bfloat16), fn.astype(jnp.bfloat16),
        (((1,), (1,)), ((), ())),
        preferred_element_type=jnp.float32), 0.0)

    scores = (p * sim) * ts

    work = scores
    for _ in range(TOP_K - 1):
        mx = jnp.max(work, axis=0, keepdims=True)
        work = jnp.where(work >= mx, -1.0, work)
    kth = jnp.max(work, axis=0, keepdims=True)

    masked = jnp.where(scores >= kth, scores, 0.0)
    w = masked / (jnp.sum(masked, axis=0, keepdims=True) + 1e-9)
    out_ref[pl.ds(row0, f.shape[0]), :] = w.T


@jax.jit
def kernel(features, W_proj, b_proj, expert_emb, expert_features, trust,
           staleness_dt):
    stale = jnp.maximum(jnp.exp(-STALENESS_LAMBDA * staleness_dt),
                        STALENESS_FLOOR)
    ts2 = (trust * stale).reshape(NUM_EXPERTS, 1)
    b2 = b_proj.reshape(1, HIDDEN_DIM)
    n_blocks = N_TOKENS // BLOCK_B
    return pl.pallas_call(
        _router_kernel,
        grid=(n_blocks,),
        in_specs=[
            pl.BlockSpec((BLOCK_B, FEATURE_DIM), lambda i: (i, 0)),
            pl.BlockSpec((FEATURE_DIM, HIDDEN_DIM), lambda i: (0, 0)),
            pl.BlockSpec((1, HIDDEN_DIM), lambda i: (0, 0)),
            pl.BlockSpec((NUM_EXPERTS, HIDDEN_DIM), lambda i: (0, 0)),
            pl.BlockSpec((NUM_EXPERTS, FEATURE_DIM), lambda i: (0, 0)),
            pl.BlockSpec((NUM_EXPERTS, 1), lambda i: (0, 0)),
        ],
        out_specs=pl.BlockSpec((BLOCK_B, NUM_EXPERTS), lambda i: (i, 0)),
        out_shape=jax.ShapeDtypeStruct((N_TOKENS, NUM_EXPERTS), jnp.float32),
        compiler_params=pltpu.CompilerParams(
            dimension_semantics=("parallel",)),
    )(features, W_proj, b2, expert_emb, expert_features, ts2)
